# baseline (speedup 1.0000x reference)
import functools

import jax
import jax.numpy as jnp
from jax import lax
from jax.experimental import pallas as pl
from jax.experimental.pallas import tpu as pltpu
from jax.experimental.pallas import tpu_sc as plsc

NUM_EMB = 100000
SEM_DIM = 4
EMB_DIM = 64
PAD = NUM_EMB * SEM_DIM
MAXV = PAD - 1
B = 1024
L = 800
LF = 4

NC = 2
NS = 16
W = NC * NS
G = 128
LPW = L // W
STEPS = LPW * 8
ET = EMB_DIM // 8


def _ids_from(s, t, m=None):
    t = jnp.minimum(jnp.maximum(t, jnp.int32(0)), jnp.int32(SEM_DIM - 1))
    ids = t * jnp.int32(NUM_EMB) + s
    inv = ((ids > jnp.int32(MAXV)) | (ids < jnp.int32(0))) & (s != jnp.int32(-1))
    ids = jnp.where(inv, jnp.int32(PAD), ids)
    if m is not None:
        ids = jnp.where(m != jnp.int32(0), ids, jnp.int32(PAD))
    return ids


def _transpose_block(rows, t5):
    iota = lax.iota(jnp.int32, 16)
    for k in range(8):
        ridx = iota + jnp.int32(k * 16)
        for et in range(ET):
            for e_in in range(8):
                e = et * 8 + e_in
                col = jnp.full((16,), e, jnp.int32)
                v = plsc.load_gather(rows, [ridx, col])
                t5[et, e_in, pl.ds(k * 16, 16)] = v


@functools.partial(
    pl.kernel,
    mesh=plsc.VectorSubcoreMesh(core_axis_name="c", subcore_axis_name="s"),
    compiler_params=pltpu.CompilerParams(use_tc_tiling_on_sc=False,
                                         needs_layout_passes=False),
    out_type=(
        jax.ShapeDtypeStruct((L, 8, 8, 8, 128), jnp.float32),
        jax.ShapeDtypeStruct((LF, 8, 8, 8, 128), jnp.float32),
    ),
    scratch_types=[
        pltpu.VMEM((2, G), jnp.int32),
        pltpu.VMEM((2, G), jnp.int32),
        pltpu.VMEM((2, G), jnp.int32),
        pltpu.VMEM((G,), jnp.int32),
        pltpu.VMEM((G,), jnp.int32),
        pltpu.VMEM((G, EMB_DIM), jnp.float32),
        pltpu.VMEM((G, EMB_DIM), jnp.float32),
        pltpu.VMEM((ET, 8, G), jnp.float32),
        pltpu.VMEM((ET, 8, G), jnp.float32),
        pltpu.SemaphoreType.DMA,
        pltpu.SemaphoreType.DMA,
        pltpu.SemaphoreType.DMA,
        pltpu.SemaphoreType.DMA,
        pltpu.SemaphoreType.DMA,
    ],
)
def _sc_embed(emb, sem, tt, msk, semf, ttf, out5, outf5,
              sv, tv, mv, idx0, idx1, rows0, rows1, t50, t51,
              lsem, g0sem, g1sem, s0sem, s1sem):
    wid = lax.axis_index("s") * NC + lax.axis_index("c")
    base = wid * (STEPS * G)
    idx = (idx0, idx1)
    rows = (rows0, rows1)
    t5 = (t50, t51)
    gsem = (g0sem, g1sem)
    ssem = (s0sem, s1sem)

    def fire_load(s, p):
        off = base + s * G
        pltpu.async_copy(sem.at[pl.ds(off, G)], sv.at[p], lsem)
        pltpu.async_copy(tt.at[pl.ds(off, G)], tv.at[p], lsem)
        pltpu.async_copy(msk.at[pl.ds(off, G)], mv.at[p], lsem)

    def wait_load(s, p):
        off = base + s * G
        pltpu.make_async_copy(sem.at[pl.ds(off, G)], sv.at[p], lsem).wait()
        pltpu.make_async_copy(tt.at[pl.ds(off, G)], tv.at[p], lsem).wait()
        pltpu.make_async_copy(msk.at[pl.ds(off, G)], mv.at[p], lsem).wait()

    def compute_idx(p):
        for k in range(G // 16):
            ids = _ids_from(sv[p, pl.ds(k * 16, 16)], tv[p, pl.ds(k * 16, 16)],
                            mv[p, pl.ds(k * 16, 16)])
            idx[p][pl.ds(k * 16, 16)] = ids

    def fire_store(s, p):
        li = wid * LPW + s // 8
        bt = s % 8
        for et in range(ET):
            pltpu.async_copy(t5[p].at[et], out5.at[li].at[et].at[bt], ssem[p])

    def wait_store(s, p):
        li = wid * LPW + s // 8
        bt = s % 8
        for et in range(ET):
            pltpu.make_async_copy(t5[p].at[et], out5.at[li].at[et].at[bt],
                                  ssem[p]).wait()

    fire_load(0, 0)

    def step(t, carry):
        for p in (0, 1):
            s = 2 * t + p
            wait_load(s, p)
            compute_idx(p)
            if p == 0:
                fire_load(s + 1, 1)
            else:
                @pl.when(t < STEPS // 2 - 1)
                def _():
                    fire_load(s + 1, 0)
            @pl.when(t > 0)
            def _():
                wait_store(s - 2, p)
            pltpu.async_copy(emb.at[idx[p]], rows[p], gsem[p])
            if p == 0:
                @pl.when(t > 0)
                def _():
                    pltpu.make_async_copy(emb.at[idx[1]], rows[1], gsem[1]).wait()
                    _transpose_block(rows[1], t5[1])
                    fire_store(s - 1, 1)
            else:
                pltpu.make_async_copy(emb.at[idx[0]], rows[0], gsem[0]).wait()
                _transpose_block(rows[0], t5[0])
                fire_store(s - 1, 0)
        return carry

    lax.fori_loop(0, STEPS // 2, step, 0)

    wait_store(STEPS - 2, 0)
    pltpu.make_async_copy(emb.at[idx[1]], rows[1], gsem[1]).wait()
    _transpose_block(rows[1], t5[1])
    fire_store(STEPS - 1, 1)

    lf = wid // 8
    btf = wid % 8
    foff = lf * B + btf * G
    pltpu.sync_copy(semf.at[pl.ds(foff, G)], sv.at[0])
    pltpu.sync_copy(ttf.at[pl.ds(foff, G)], tv.at[0])
    for k in range(G // 16):
        ids = _ids_from(sv[0, pl.ds(k * 16, 16)], tv[0, pl.ds(k * 16, 16)])
        idx[0][pl.ds(k * 16, 16)] = ids
    pltpu.async_copy(emb.at[idx[0]], rows[0], gsem[0])
    pltpu.make_async_copy(emb.at[idx[0]], rows[0], gsem[0]).wait()
    _transpose_block(rows[0], t50)
    for et in range(ET):
        pltpu.sync_copy(t50.at[et], outf5.at[lf].at[et].at[btf])

    wait_store(STEPS - 1, 1)


def kernel(emb, sem_ids, token_type_ids, seq_mask, sem_ids_fut, token_type_ids_fut):
    sem = sem_ids.T.reshape(-1)
    tt = token_type_ids.T.reshape(-1)
    msk = seq_mask.T.astype(jnp.int32).reshape(-1)
    semf = sem_ids_fut.T.reshape(-1)
    ttf = token_type_ids_fut.T.reshape(-1)
    out5, outf5 = _sc_embed(emb, sem, tt, msk, semf, ttf)
    out = out5.transpose((2, 4, 0, 1, 3)).reshape(B, L, EMB_DIM)
    outf = outf5.transpose((2, 4, 0, 1, 3)).reshape(B, LF, EMB_DIM)
    return out, outf


# speedup vs baseline: 1.2125x; 1.2125x over previous
import functools

import jax
import jax.numpy as jnp
from jax import lax
from jax.experimental import pallas as pl
from jax.experimental.pallas import tpu as pltpu
from jax.experimental.pallas import tpu_sc as plsc

NUM_EMB = 100000
SEM_DIM = 4
EMB_DIM = 64
PAD = NUM_EMB * SEM_DIM
MAXV = PAD - 1
B = 1024
L = 800
LF = 4

NC = 2
NS = 16
W = NC * NS
G = 128
LPW = L // W
STEPS = LPW * 8
ET = EMB_DIM // 8


def _ids_from(s, t, m=None):
    t = jnp.minimum(jnp.maximum(t, jnp.int32(0)), jnp.int32(SEM_DIM - 1))
    ids = t * jnp.int32(NUM_EMB) + s
    inv = ((ids > jnp.int32(MAXV)) | (ids < jnp.int32(0))) & (s != jnp.int32(-1))
    ids = jnp.where(inv, jnp.int32(PAD), ids)
    if m is not None:
        ids = jnp.where(m != jnp.int32(0), ids, jnp.int32(PAD))
    return ids


def _transpose_block(rows, t5):
    iota = lax.iota(jnp.int32, 16)
    ridx = [iota + jnp.int32(k * 16) for k in range(8)]
    for et in range(ET):
        for e_in in range(8):
            e = et * 8 + e_in
            col = jnp.full((16,), e, jnp.int32)
            vs = [plsc.load_gather(rows, [ridx[k], col]) for k in range(8)]
            for k in range(8):
                t5[et, e_in, pl.ds(k * 16, 16)] = vs[k]


@functools.partial(
    pl.kernel,
    mesh=plsc.VectorSubcoreMesh(core_axis_name="c", subcore_axis_name="s"),
    compiler_params=pltpu.CompilerParams(use_tc_tiling_on_sc=False,
                                         needs_layout_passes=False,
                                         disable_bounds_checks=True),
    out_type=(
        jax.ShapeDtypeStruct((L, 8, 8, 8, 128), jnp.float32),
        jax.ShapeDtypeStruct((LF, 8, 8, 8, 128), jnp.float32),
    ),
    scratch_types=[
        pltpu.VMEM((2, G), jnp.int32),
        pltpu.VMEM((2, G), jnp.int32),
        pltpu.VMEM((2, G), jnp.int32),
        pltpu.VMEM((G,), jnp.int32),
        pltpu.VMEM((G,), jnp.int32),
        pltpu.VMEM((G, EMB_DIM), jnp.float32),
        pltpu.VMEM((G, EMB_DIM), jnp.float32),
        pltpu.VMEM((ET, 8, G), jnp.float32),
        pltpu.VMEM((ET, 8, G), jnp.float32),
        pltpu.SemaphoreType.DMA,
        pltpu.SemaphoreType.DMA,
        pltpu.SemaphoreType.DMA,
        pltpu.SemaphoreType.DMA,
        pltpu.SemaphoreType.DMA,
    ],
)
def _sc_embed(emb, sem, tt, msk, semf, ttf, out5, outf5,
              sv, tv, mv, idx0, idx1, rows0, rows1, t50, t51,
              lsem, g0sem, g1sem, s0sem, s1sem):
    wid = lax.axis_index("s") * NC + lax.axis_index("c")
    base = wid * (STEPS * G)
    idx = (idx0, idx1)
    rows = (rows0, rows1)
    t5 = (t50, t51)
    gsem = (g0sem, g1sem)
    ssem = (s0sem, s1sem)

    def fire_load(s, p):
        off = base + s * G
        pltpu.async_copy(sem.at[pl.ds(off, G)], sv.at[p], lsem)
        pltpu.async_copy(tt.at[pl.ds(off, G)], tv.at[p], lsem)
        pltpu.async_copy(msk.at[pl.ds(off, G)], mv.at[p], lsem)

    def wait_load(s, p):
        off = base + s * G
        pltpu.make_async_copy(sem.at[pl.ds(off, G)], sv.at[p], lsem).wait()
        pltpu.make_async_copy(tt.at[pl.ds(off, G)], tv.at[p], lsem).wait()
        pltpu.make_async_copy(msk.at[pl.ds(off, G)], mv.at[p], lsem).wait()

    def compute_idx(p):
        for k in range(G // 16):
            ids = _ids_from(sv[p, pl.ds(k * 16, 16)], tv[p, pl.ds(k * 16, 16)],
                            mv[p, pl.ds(k * 16, 16)])
            idx[p][pl.ds(k * 16, 16)] = ids

    def fire_store(s, p):
        li = wid * LPW + s // 8
        bt = s % 8
        for et in range(ET):
            pltpu.async_copy(t5[p].at[et], out5.at[li].at[et].at[bt], ssem[p])

    def wait_store(s, p):
        li = wid * LPW + s // 8
        bt = s % 8
        for et in range(ET):
            pltpu.make_async_copy(t5[p].at[et], out5.at[li].at[et].at[bt],
                                  ssem[p]).wait()

    fire_load(0, 0)

    def step(t, carry):
        for p in (0, 1):
            s = 2 * t + p
            wait_load(s, p)
            compute_idx(p)
            if p == 0:
                fire_load(s + 1, 1)
            else:
                @pl.when(t < STEPS // 2 - 1)
                def _():
                    fire_load(s + 1, 0)
            @pl.when(t > 0)
            def _():
                wait_store(s - 2, p)
            pltpu.async_copy(emb.at[idx[p]], rows[p], gsem[p])
            if p == 0:
                @pl.when(t > 0)
                def _():
                    pltpu.make_async_copy(emb.at[idx[1]], rows[1], gsem[1]).wait()
                    _transpose_block(rows[1], t5[1])
                    fire_store(s - 1, 1)
            else:
                pltpu.make_async_copy(emb.at[idx[0]], rows[0], gsem[0]).wait()
                _transpose_block(rows[0], t5[0])
                fire_store(s - 1, 0)
        return carry

    lax.fori_loop(0, STEPS // 2, step, 0)

    wait_store(STEPS - 2, 0)
    pltpu.make_async_copy(emb.at[idx[1]], rows[1], gsem[1]).wait()
    _transpose_block(rows[1], t5[1])
    fire_store(STEPS - 1, 1)

    lf = wid // 8
    btf = wid % 8
    foff = lf * B + btf * G
    pltpu.sync_copy(semf.at[pl.ds(foff, G)], sv.at[0])
    pltpu.sync_copy(ttf.at[pl.ds(foff, G)], tv.at[0])
    for k in range(G // 16):
        ids = _ids_from(sv[0, pl.ds(k * 16, 16)], tv[0, pl.ds(k * 16, 16)])
        idx[0][pl.ds(k * 16, 16)] = ids
    pltpu.async_copy(emb.at[idx[0]], rows[0], gsem[0])
    pltpu.make_async_copy(emb.at[idx[0]], rows[0], gsem[0]).wait()
    _transpose_block(rows[0], t50)
    for et in range(ET):
        pltpu.sync_copy(t50.at[et], outf5.at[lf].at[et].at[btf])

    wait_store(STEPS - 1, 1)


def kernel(emb, sem_ids, token_type_ids, seq_mask, sem_ids_fut, token_type_ids_fut):
    sem = sem_ids.T.reshape(-1)
    tt = token_type_ids.T.reshape(-1)
    msk = seq_mask.T.astype(jnp.int32).reshape(-1)
    semf = sem_ids_fut.T.reshape(-1)
    ttf = token_type_ids_fut.T.reshape(-1)
    out5, outf5 = _sc_embed(emb, sem, tt, msk, semf, ttf)
    out = out5.transpose((2, 4, 0, 1, 3)).reshape(B, L, EMB_DIM)
    outf = outf5.transpose((2, 4, 0, 1, 3)).reshape(B, LF, EMB_DIM)
    return out, outf
